# baseline (speedup 1.0000x reference)
import functools

import jax
import jax.numpy as jnp
from jax import lax
from jax.experimental import pallas as pl
from jax.experimental.pallas import tpu as pltpu


def _fused(src_ref, w_ref, bias_ref, edge_ref, sid_ref, didp_ref, didc_ref,
           out_ref, sproj_ref, acc_ref, msg_ref,
           *, ns, nd, fo, fi, te, n_body):
    step = pl.program_id(0)
    bf16 = jnp.bfloat16

    @pl.when(step == 0)
    def _():
        sp_t = lax.dot_general(w_ref[0:fi, :].astype(bf16),
                               src_ref[...].astype(bf16),
                               (((0,), (1,)), ((), ())),
                               preferred_element_type=jnp.float32)
        sproj_ref[...] = sp_t.astype(bf16)
        msg_ref[...] = jnp.zeros_like(msg_ref)

    def _oh_dst(did_row):
        return (lax.broadcasted_iota(jnp.int32, (te, nd), 1)
                == did_row.T).astype(bf16)

    def _message(sid, edge, w_edge):
        oh_src = (lax.broadcasted_iota(jnp.int32, (ns, te), 0)
                  == sid).astype(bf16)
        gath_t = lax.dot_general(sproj_ref[...], oh_src,
                                 (((1,), (0,)), ((), ())),
                                 preferred_element_type=jnp.float32)
        ep_t = lax.dot_general(w_edge, edge.astype(bf16),
                               (((0,), (1,)), ((), ())),
                               preferred_element_type=jnp.float32)
        return jnp.concatenate(
            [(gath_t + ep_t).astype(bf16), jnp.ones((8, te), bf16)], axis=0)

    def _scatter(msg_t, did_row):
        return lax.dot_general(msg_t, _oh_dst(did_row),
                               (((1,), (0,)), ((), ())),
                               preferred_element_type=jnp.float32)

    @pl.when(step < n_body)
    def _():
        w_edge = w_ref[fi:2 * fi, :].astype(bf16)
        contrib_prev = _scatter(msg_ref[...], didp_ref[...])
        msg_a = _message(sid_ref[:, 0:te], edge_ref[0:te, :], w_edge)
        contrib_a = _scatter(msg_a, didc_ref[...])
        msg_b = _message(sid_ref[:, te:2 * te], edge_ref[te:2 * te, :],
                         w_edge)
        both = contrib_prev + contrib_a
        acc_ref[...] = jnp.where(step == 0, both, acc_ref[...] + both)
        msg_ref[...] = msg_b

    @pl.when(step == n_body)
    def _():
        acc = acc_ref[...] + _scatter(msg_ref[...], didp_ref[...])
        deg = acc[fo:fo + 1, :]
        norm = jnp.where(deg > 0, lax.rsqrt(deg), 0.0)
        out_t = acc[0:fo, :] * norm + bias_ref[...].T
        out_ref[...] = out_t.T


def kernel(src_feats, edge_feats, src_ids, dst_ids, weights, bias,
           n_dst=2048, te=4096):
    f32 = jnp.float32
    bf16 = jnp.bfloat16
    n_src, in_feat = src_feats.shape
    n_edges = edge_feats.shape[0]
    out_feat = weights.shape[1]

    assert n_edges % (2 * te) == 0
    n_body = n_edges // (2 * te)
    n_sub = n_edges // te
    m = out_feat + 8

    sid_row = src_ids.astype(jnp.int32).reshape(1, n_edges)
    did_row = dst_ids.astype(jnp.int32).reshape(1, n_edges)
    bias_row = bias.astype(f32).reshape(1, out_feat)

    out = pl.pallas_call(
        functools.partial(_fused, ns=n_src, nd=n_dst, fo=out_feat,
                          fi=in_feat, te=te, n_body=n_body),
        grid=(n_body + 1,),
        in_specs=[
            pl.BlockSpec((n_src, in_feat), lambda s: (0, 0)),
            pl.BlockSpec((2 * in_feat, out_feat), lambda s: (0, 0)),
            pl.BlockSpec((1, out_feat), lambda s: (0, 0)),
            pl.BlockSpec((2 * te, in_feat),
                         lambda s: (jnp.minimum(s, n_body - 1), 0)),
            pl.BlockSpec((1, 2 * te),
                         lambda s: (0, jnp.minimum(s, n_body - 1))),
            pl.BlockSpec((1, te),
                         lambda s: (0, jnp.clip(2 * s - 1, 0, n_sub - 1))),
            pl.BlockSpec((1, te),
                         lambda s: (0, jnp.minimum(2 * s, n_sub - 1))),
        ],
        out_specs=pl.BlockSpec((n_dst, out_feat), lambda s: (0, 0)),
        out_shape=jax.ShapeDtypeStruct((n_dst, out_feat), f32),
        scratch_shapes=[
            pltpu.VMEM((out_feat, n_src), bf16),
            pltpu.VMEM((m, n_dst), f32),
            pltpu.VMEM((m, te), bf16),
        ],
        compiler_params=pltpu.CompilerParams(
            dimension_semantics=("arbitrary",),
            vmem_limit_bytes=100 * 1024 * 1024),
    )(src_feats, weights.astype(f32), bias_row, edge_feats,
      sid_row, did_row, did_row)

    return out


# speedup vs baseline: 1.0339x; 1.0339x over previous
import functools

import jax
import jax.numpy as jnp
from jax import lax
from jax.experimental import pallas as pl
from jax.experimental.pallas import tpu as pltpu


def _fused(src_ref, w_ref, bias_ref, edge_ref,
           sid_ref, did_ref, out_ref, sproj_ref, acc_ref,
           *, ns, nd, fo, fi, te, n_tiles):
    step = pl.program_id(0)

    @pl.when(step == 0)
    def _():
        sp_t = lax.dot_general(w_ref[0:fi, :].astype(jnp.bfloat16),
                               src_ref[...].astype(jnp.bfloat16),
                               (((0,), (1,)), ((), ())),
                               preferred_element_type=jnp.float32)
        sproj_ref[...] = sp_t.astype(jnp.bfloat16)

    w_edge = w_ref[fi:2 * fi, :].astype(jnp.bfloat16)

    def _tile(sid, did, edge):
        oh_src = (lax.broadcasted_iota(jnp.int32, (ns, te), 0)
                  == sid).astype(jnp.bfloat16)
        gath_t = lax.dot_general(sproj_ref[...], oh_src,
                                 (((1,), (0,)), ((), ())),
                                 preferred_element_type=jnp.float32)
        ep_t = lax.dot_general(w_edge, edge.astype(jnp.bfloat16),
                               (((0,), (1,)), ((), ())),
                               preferred_element_type=jnp.float32)
        msg_t = jnp.concatenate(
            [(gath_t + ep_t).astype(jnp.bfloat16),
             jnp.ones((8, te), jnp.bfloat16)], axis=0)
        oh_dst = (lax.broadcasted_iota(jnp.int32, (te, nd), 1)
                  == did.T).astype(jnp.bfloat16)
        return lax.dot_general(msg_t, oh_dst, (((1,), (0,)), ((), ())),
                               preferred_element_type=jnp.float32)

    contrib_a = _tile(sid_ref[:, 0:te], did_ref[:, 0:te],
                      edge_ref[0:te, :])
    contrib_b = _tile(sid_ref[:, te:2 * te], did_ref[:, te:2 * te],
                      edge_ref[te:2 * te, :])
    contrib = contrib_a + contrib_b

    @pl.when(step == 0)
    def _():
        acc_ref[...] = contrib

    @pl.when(step != 0)
    def _():
        acc_ref[...] += contrib

    @pl.when(step == n_tiles - 1)
    def _():
        acc = acc_ref[0:fo, :]
        deg = acc_ref[fo:fo + 1, :]
        norm = jnp.where(deg > 0, lax.rsqrt(deg), 0.0)
        out_t = acc * norm + bias_ref[...].T
        out_ref[...] = out_t.T


def kernel(src_feats, edge_feats, src_ids, dst_ids, weights, bias,
           n_dst=2048, te=4096):
    f32 = jnp.float32
    bf16 = jnp.bfloat16
    n_src, in_feat = src_feats.shape
    n_edges = edge_feats.shape[0]
    out_feat = weights.shape[1]

    assert n_edges % (2 * te) == 0
    n_tiles = n_edges // (2 * te)
    m = out_feat + 8

    sid_row = src_ids.astype(jnp.int32).reshape(1, n_edges)
    did_row = dst_ids.astype(jnp.int32).reshape(1, n_edges)
    bias_row = bias.astype(f32).reshape(1, out_feat)

    out = pl.pallas_call(
        functools.partial(_fused, ns=n_src, nd=n_dst, fo=out_feat,
                          fi=in_feat, te=te, n_tiles=n_tiles),
        grid=(n_tiles,),
        in_specs=[
            pl.BlockSpec((n_src, in_feat), lambda e: (0, 0)),
            pl.BlockSpec((2 * in_feat, out_feat), lambda e: (0, 0)),
            pl.BlockSpec((1, out_feat), lambda e: (0, 0)),
            pl.BlockSpec((2 * te, in_feat), lambda e: (e, 0)),
            pl.BlockSpec((1, 2 * te), lambda e: (0, e)),
            pl.BlockSpec((1, 2 * te), lambda e: (0, e)),
        ],
        out_specs=pl.BlockSpec((n_dst, out_feat), lambda e: (0, 0)),
        out_shape=jax.ShapeDtypeStruct((n_dst, out_feat), f32),
        scratch_shapes=[
            pltpu.VMEM((out_feat, n_src), bf16),
            pltpu.VMEM((m, n_dst), f32),
        ],
        compiler_params=pltpu.CompilerParams(
            dimension_semantics=("arbitrary",),
            vmem_limit_bytes=100 * 1024 * 1024),
    )(src_feats, weights.astype(f32), bias_row, edge_feats, sid_row, did_row)

    return out
